# baseline (speedup 1.0000x reference)
import jax
import jax.numpy as jnp
from jax import lax
from jax.experimental import pallas as pl
from jax.experimental.pallas import tpu as pltpu
from jax.experimental.pallas import tpu_sc as plsc

_DIM = 64
_BW = 128
_LB = 8
_NW = 32
_NBLK = 25


def _gather_body(tbl2, idx_hbm, out_hbm,
                 idxv, idx2v, hv64v, gbuf, sbuf, dump,
                 gsem0, gsem1, osem0, osem1):
    gbufs = (gbuf.at[0], gbuf.at[1])
    sbufs = (sbuf.at[0], sbuf.at[1])
    gsems = (gsem0, gsem1)
    osems = (osem0, osem1)

    wid = lax.axis_index("s") * 2 + lax.axis_index("c")
    b0 = wid * _BW

    def load_idx_block(lb):
        pltpu.sync_copy(idx_hbm.at[pl.ds(lb * _LB, _LB), pl.ds(b0, _BW)], idxv)
        def prep(g, carry):
            r = g // 8
            c = (g % 8) * 16
            v = idxv[r, pl.ds(c, 16)]
            idx2v[r, pl.ds(c, 16)] = lax.shift_right_logical(v, 1)
            hv64v[r, pl.ds(c, 16)] = lax.shift_left(
                lax.bitwise_and(v, 1), 6)
            return carry
        lax.fori_loop(0, _LB * 8, prep, 0)

    def fire_gather(r, buf):
        pltpu.async_copy(tbl2.at[idx2v.at[r]], gbufs[buf], gsems[buf])

    def wait_gather(buf):
        pltpu.make_async_copy(tbl2.at[idx2v.at[0]], gbufs[buf],
                              gsems[buf]).wait()

    def fire_out(r, lb, buf):
        pltpu.async_copy(sbufs[buf],
                         out_hbm.at[lb * _LB + r, :, pl.ds(b0, _BW)],
                         osems[buf])

    def wait_out(buf):
        pltpu.make_async_copy(sbufs[buf], dump, osems[buf]).wait()

    def transpose_row(r, buf):
        lane = lax.iota(jnp.int32, 16)
        for g in range(8):
            rows = lane + (16 * g)
            hv = hv64v[r, pl.ds(16 * g, 16)]
            def dstep(d, carry):
                v = plsc.load_gather(gbufs[buf], [rows, hv + d])
                sbufs[buf][d, pl.ds(16 * g, 16)] = v
                return carry
            lax.fori_loop(0, _DIM, dstep, 0)

    pltpu.async_copy(sbufs[0], dump, osems[0])
    pltpu.async_copy(sbufs[1], dump, osems[1])

    def block(lb, carry):
        load_idx_block(lb)
        fire_gather(0, 0)
        fire_gather(1, 1)
        for r in range(_LB):
            buf = r % 2
            wait_gather(buf)
            wait_out(buf)
            transpose_row(r, buf)
            fire_out(r, lb, buf)
            if r + 2 < _LB:
                fire_gather(r + 2, buf)
        return carry

    lax.fori_loop(0, _NBLK, block, 0)
    wait_out(0)
    wait_out(1)


def kernel(inputs, table):
    seq, batch = inputs.shape
    vocab = table.shape[0]
    tbl2 = table.reshape(vocab // 2, 2 * _DIM)
    mesh = plsc.VectorSubcoreMesh(core_axis_name="c", subcore_axis_name="s")
    out_t = pl.kernel(
        _gather_body,
        out_type=jax.ShapeDtypeStruct((seq, _DIM, batch), jnp.float32),
        mesh=mesh,
        compiler_params=pltpu.CompilerParams(needs_layout_passes=False),
        scratch_types=[
            pltpu.VMEM((_LB, _BW), jnp.int32),
            pltpu.VMEM((_LB, _BW), jnp.int32),
            pltpu.VMEM((_LB, _BW), jnp.int32),
            pltpu.VMEM((2, _BW, 2 * _DIM), jnp.float32),
            pltpu.VMEM((2, _DIM, _BW), jnp.float32),
            pltpu.HBM((_DIM, _BW), jnp.float32),
            pltpu.SemaphoreType.DMA,
            pltpu.SemaphoreType.DMA,
            pltpu.SemaphoreType.DMA,
            pltpu.SemaphoreType.DMA,
        ],
    )(tbl2, inputs)
    return out_t.transpose(0, 2, 1)
